# baseline (speedup 1.0000x reference)
import functools

import jax
import jax.numpy as jnp
from jax import lax
from jax.experimental import pallas as pl
from jax.experimental.pallas import tpu as pltpu
from jax.experimental.pallas import tpu_sc as plsc

NUM_CODES = 8192
DIM = 32
NUM_TOKENS = 8192
TOKEN_TILE = 256
NUM_TILES = NUM_TOKENS // TOKEN_TILE

SC_NC = 2
SC_NS = 16
SC_NW = SC_NC * SC_NS
TOK_PER_W = NUM_TOKENS // SC_NW


def _argmin_body(x_ref, wt_ref, xn_ref, idx_ref):
    xt = x_ref[...]
    wt = wt_ref[...]
    xn = xn_ref[0, 0, :][:, None]
    wn = jnp.sum(wt * wt, axis=0, keepdims=True)
    prod = lax.dot_general(xt, wt, (((1,), (0,)), ((), ())),
                           preferred_element_type=jnp.float32,
                           precision=lax.Precision.HIGHEST)
    sq = (xn - 2.0 * prod) + wn
    dist = jnp.sqrt(jnp.maximum(sq, 0.0))
    minv = jnp.min(dist, axis=1, keepdims=True)
    cols = lax.broadcasted_iota(jnp.int32, dist.shape, 1)
    idx = jnp.min(jnp.where(dist == minv, cols, jnp.int32(NUM_CODES)), axis=1)
    idx_ref[0, 0, :] = idx


def _nearest_code_indices(x_flat, weight):
    xn = jnp.sum(x_flat * x_flat, axis=1, keepdims=True)
    xn3 = xn.reshape(NUM_TILES, 1, TOKEN_TILE)
    out = pl.pallas_call(
        _argmin_body,
        grid=(NUM_TILES,),
        in_specs=[
            pl.BlockSpec((TOKEN_TILE, DIM), lambda i: (i, 0)),
            pl.BlockSpec((DIM, NUM_CODES), lambda i: (0, 0)),
            pl.BlockSpec((1, 1, TOKEN_TILE), lambda i: (i, 0, 0)),
        ],
        out_specs=pl.BlockSpec((1, 1, TOKEN_TILE), lambda i: (i, 0, 0)),
        out_shape=jax.ShapeDtypeStruct((NUM_TILES, 1, TOKEN_TILE), jnp.int32),
    )(x_flat, weight.T, xn3)
    return out.reshape(NUM_TOKENS)


def _sc_gather_body(w_hbm, idx_hbm, out_hbm, idx_v, rows_v, sem):
    wid = lax.axis_index("s") * SC_NC + lax.axis_index("c")
    base = wid * TOK_PER_W
    pltpu.sync_copy(idx_hbm.at[pl.ds(base, TOK_PER_W)], idx_v)
    pltpu.async_copy(w_hbm.at[idx_v], rows_v, sem).wait()
    pltpu.sync_copy(rows_v, out_hbm.at[pl.ds(base, TOK_PER_W)])


def _sc_gather(weight, idx):
    mesh = plsc.VectorSubcoreMesh(core_axis_name="c", subcore_axis_name="s")
    fn = functools.partial(
        pl.kernel,
        mesh=mesh,
        out_type=jax.ShapeDtypeStruct((NUM_TOKENS, DIM), jnp.float32),
        scratch_types=[
            pltpu.VMEM((TOK_PER_W,), jnp.int32),
            pltpu.VMEM((TOK_PER_W, DIM), jnp.float32),
            pltpu.SemaphoreType.DMA,
        ],
        compiler_params=pltpu.CompilerParams(use_tc_tiling_on_sc=False),
    )(_sc_gather_body)
    return fn(weight, idx)


def kernel(x, weight):
    B, C, H, W = x.shape
    x_flat = jnp.transpose(x, (0, 2, 3, 1)).reshape(-1, C)
    idx = _nearest_code_indices(x_flat, weight)
    quantized = _sc_gather(weight, idx)
    return jnp.transpose(quantized.reshape(B, H, W, C), (0, 3, 1, 2))
